# baseline (speedup 1.0000x reference)
import functools

import jax
import jax.numpy as jnp
from jax import lax
from jax.experimental import pallas as pl
from jax.experimental.pallas import tpu as pltpu
from jax.experimental.pallas import tpu_sc as plsc

TOTAL = 1000000
NUM_EMB = TOTAL - 2
PAD = NUM_EMB - 1
START = NUM_EMB
DIM = 32
BATCH = 4096
HIST = 200

NC, NS, L = 2, 16, 16
NW = NC * NS
BB = BATCH // NW
HC = 8
NCHUNK = HIST // HC
CHUNK = BB * HC
NBG = BB // L
NGRP = CHUNK // L

TCOLS = 999998
CW = 512
NBLK = (TCOLS // CW)
BPW = NBLK // NW + 1
TAIL0 = NBLK * CW
TAILW = TCOLS - TAIL0


def _detile_body(tt_hbm, t128_hbm, in_v, sk_v, osem):
    c = lax.axis_index("c")
    s = lax.axis_index("s")
    wid = s * NC + c
    lane = lax.iota(jnp.int32, L)
    lane32 = lane * 32

    def transpose_block():
        @plsc.parallel_loop(0, CW * (DIM // L), unroll=16)
        def _tr(t):
            c = t >> 1
            d0 = (t & 1) * L
            w0 = c * DIM + d0
            vals = plsc.load_gather(in_v, [d0 + lane,
                                           jnp.full((L,), c, jnp.int32)])
            plsc.store_scatter(sk_v, [jnp.full((L,), w0 >> 7, jnp.int32),
                                      (w0 & 127) + lane], vals)

    BR = CW * DIM // 128

    def out_copy(row0, rows):
        return pltpu.make_async_copy(
            sk_v.at[pl.ds(0, rows), pl.ds(0, 128)],
            t128_hbm.at[pl.ds(row0, rows)], osem)

    def blk(g, carry):
        bi = wid + NW * g

        @pl.when(bi < NBLK)
        def _do():
            pltpu.sync_copy(tt_hbm.at[:, pl.ds(bi * CW, CW)],
                            in_v.at[:, pl.ds(0, CW)])

            @pl.when(g > 0)
            def _drain():
                out_copy(bi * BR, BR).wait()

            transpose_block()
            out_copy(bi * BR, BR).start()
        return carry

    lax.fori_loop(0, BPW, blk, 0)
    out_copy(0, BR).wait()


def _body(idx_hbm, table_hbm, aux_hbm, out_hbm,
          ir_v, is_v, rows_v, tv_v, aux_v, gsem, osem):
    c = lax.axis_index("c")
    s = lax.axis_index("s")
    wid = s * NC + c
    b0 = wid * BB

    pltpu.sync_copy(aux_hbm, aux_v)
    lane = lax.iota(jnp.int32, L)

    def out_copy(ci):
        return pltpu.make_async_copy(
            tv_v.at[:, :, pl.ds(0, BB)],
            out_hbm.at[pl.ds(ci * HC, HC), :, pl.ds(b0, BB)], osem)

    def pass1():
        @plsc.parallel_loop(0, NGRP, unroll=8,
                            carry=jnp.zeros((L,), jnp.int32))
        def grp(g, mx):
            jv = g * L + lane
            v = plsc.load_gather(ir_v, [jv >> 3, jv & (HC - 1)])
            is_v[pl.ds(g * L, L)] = jnp.minimum(v, TAIL0 - 1)
            return jnp.maximum(mx, v)

        return grp

    def fixup(mx):
        has_special = plsc.all_reduce_population_count(mx >= TAIL0)[0] > 0

        @pl.when(has_special)
        def _fix():
            def grp_body(g, carry):
                jv = g * L + lane
                v = plsc.load_gather(ir_v, [jv >> 3, jv & (HC - 1)])
                mask = v >= TAIL0
                g_has = plsc.all_reduce_population_count(mask)[0] > 0

                @pl.when(g_has)
                def _overwrite():
                    sel = jnp.maximum(v - TAIL0, 0)
                    for col in range(DIM):
                        colv = jnp.full((L,), col, jnp.int32)
                        repl = plsc.load_gather(aux_v, [sel, colv], mask=mask)
                        plsc.store_scatter(rows_v, [jv, colv], repl,
                                           mask=mask)
                return carry

            lax.fori_loop(0, NGRP, grp_body, 0)

    def chunk_body(ci, carry):
        pltpu.sync_copy(idx_hbm.at[pl.ds(b0, BB), pl.ds(ci * HC, HC)], ir_v)
        mx = pass1()
        gathers = [
            pltpu.async_copy(
                table_hbm.at[is_v.at[pl.ds(bg * (L * HC), L * HC)]],
                rows_v.at[pl.ds(bg * (L * HC), L * HC)], gsem)
            for bg in range(NBG)
        ]
        for cp in gathers:
            cp.wait()
        fixup(mx)

        @pl.when(ci > 0)
        def _drain():
            out_copy(ci).wait()

        @plsc.parallel_loop(0, CHUNK, unroll=16)
        def _tr(j):
            b = j >> 3
            h = j & (HC - 1)
            jvec = jnp.full((L,), j, jnp.int32)
            hvec = jnp.full((L,), h, jnp.int32)
            bvec = jnp.full((L,), b, jnp.int32)
            v0 = plsc.load_gather(rows_v, [jvec, lane])
            v1 = plsc.load_gather(rows_v, [jvec, L + lane])
            plsc.store_scatter(tv_v, [hvec, lane, bvec], v0)
            plsc.store_scatter(tv_v, [hvec, L + lane, bvec], v1)

        out_copy(ci).start()
        return carry

    lax.fori_loop(0, NCHUNK, chunk_body, 0)
    out_copy(NCHUNK - 1).wait()


def _detile(table_t):
    f = functools.partial(
        pl.kernel,
        mesh=plsc.VectorSubcoreMesh(core_axis_name="c", subcore_axis_name="s"),
        out_type=jax.ShapeDtypeStruct((TOTAL * DIM // 128, 128), jnp.float32),
        scratch_types=[
            pltpu.VMEM((DIM, CW + 1), jnp.float32),
            pltpu.VMEM((CW * DIM // 128, 129), jnp.float32),
            pltpu.SemaphoreType.DMA,
        ],
        compiler_params=pltpu.CompilerParams(
            needs_layout_passes=False, use_tc_tiling_on_sc=True),
    )(_detile_body)
    return f(table_t)


@jax.jit
def _run(idxes, table, beg_end):
    t128 = _detile(table.T)
    table_lin = t128.reshape(TOTAL, DIM)
    aux = jnp.concatenate([table[TAIL0:], beg_end], axis=0)
    f = functools.partial(
        pl.kernel,
        mesh=plsc.VectorSubcoreMesh(core_axis_name="c", subcore_axis_name="s"),
        out_type=jax.ShapeDtypeStruct((HIST, DIM, BATCH), jnp.float32),
        scratch_types=[
            pltpu.VMEM((BB, HC), jnp.int32),
            pltpu.VMEM((CHUNK,), jnp.int32),
            pltpu.VMEM((CHUNK, DIM), jnp.float32),
            pltpu.VMEM((HC, DIM, BB + 1), jnp.float32),
            pltpu.VMEM((TOTAL - TAIL0, DIM), jnp.float32),
            pltpu.SemaphoreType.DMA,
            pltpu.SemaphoreType.DMA,
        ],
        compiler_params=pltpu.CompilerParams(
            needs_layout_passes=False, use_tc_tiling_on_sc=False),
    )(_body)
    return f(idxes, table_lin, aux)


def kernel(idxes, table, beg_end):
    return _run(idxes, table, beg_end).transpose(2, 0, 1)


# speedup vs baseline: 1.1773x; 1.1773x over previous
import functools

import jax
import jax.numpy as jnp
from jax import lax
from jax.experimental import pallas as pl
from jax.experimental.pallas import tpu as pltpu
from jax.experimental.pallas import tpu_sc as plsc

TOTAL = 1000000
NUM_EMB = TOTAL - 2
PAD = NUM_EMB - 1
START = NUM_EMB
DIM = 32
BATCH = 4096
HIST = 200

NC, NS, L = 2, 16, 16
NW = NC * NS
BB = BATCH // NW
HC = 8
NCHUNK = HIST // HC
CHUNK = BB * HC
NBG = BB // L
NGRP = CHUNK // L


def _body(idx_hbm, table_hbm, be_hbm, out_hbm,
          ir_v, is_v, rows_v, tv_v, be_v, gsem, osem):
    c = lax.axis_index("c")
    s = lax.axis_index("s")
    wid = s * NC + c
    b0 = wid * BB

    pltpu.sync_copy(be_hbm, be_v)
    lane = lax.iota(jnp.int32, L)

    def out_copy(ci):
        return pltpu.make_async_copy(
            tv_v.at[:, :, pl.ds(0, BB)],
            out_hbm.at[pl.ds(ci * HC, HC), :, pl.ds(b0, BB)], osem)

    def pass1():
        @plsc.parallel_loop(0, NGRP, unroll=8,
                            carry=jnp.zeros((L,), jnp.int32))
        def grp(g, mx):
            jv = g * L + lane
            v = plsc.load_gather(ir_v, [jv >> 3, jv & (HC - 1)])
            is_v[pl.ds(g * L, L)] = jnp.minimum(v, PAD)
            return jnp.maximum(mx, v)

        return grp

    def fixup(mx):
        has_special = plsc.all_reduce_population_count(mx >= START)[0] > 0

        @pl.when(has_special)
        def _fix():
            def grp_body(g, carry):
                jv = g * L + lane
                v = plsc.load_gather(ir_v, [jv >> 3, jv & (HC - 1)])
                mask = v >= START
                g_has = plsc.all_reduce_population_count(mask)[0] > 0

                @pl.when(g_has)
                def _overwrite():
                    sel = jnp.clip(v - START, 0, 1)
                    for col in range(DIM):
                        colv = jnp.full((L,), col, jnp.int32)
                        repl = plsc.load_gather(be_v, [sel, colv], mask=mask)
                        plsc.store_scatter(rows_v, [jv, colv], repl,
                                           mask=mask)
                return carry

            lax.fori_loop(0, NGRP, grp_body, 0)

    def chunk_body(ci, carry):
        pltpu.sync_copy(idx_hbm.at[pl.ds(b0, BB), pl.ds(ci * HC, HC)], ir_v)
        mx = pass1()
        gathers = [
            pltpu.async_copy(
                table_hbm.at[is_v.at[pl.ds(bg * (L * HC), L * HC)]],
                rows_v.at[pl.ds(bg * (L * HC), L * HC)], gsem)
            for bg in range(NBG)
        ]
        for cp in gathers:
            cp.wait()
        fixup(mx)

        @pl.when(ci > 0)
        def _drain():
            out_copy(ci).wait()

        @plsc.parallel_loop(0, CHUNK, unroll=16)
        def _tr(j):
            b = j >> 3
            h = j & (HC - 1)
            jvec = jnp.full((L,), j, jnp.int32)
            hvec = jnp.full((L,), h, jnp.int32)
            bvec = jnp.full((L,), b, jnp.int32)
            v0 = plsc.load_gather(rows_v, [jvec, lane])
            v1 = plsc.load_gather(rows_v, [jvec, L + lane])
            plsc.store_scatter(tv_v, [hvec, lane, bvec], v0)
            plsc.store_scatter(tv_v, [hvec, L + lane, bvec], v1)

        out_copy(ci).start()
        return carry

    lax.fori_loop(0, NCHUNK, chunk_body, 0)
    out_copy(NCHUNK - 1).wait()


@jax.jit
def _run(idxes, table, beg_end):
    f = functools.partial(
        pl.kernel,
        mesh=plsc.VectorSubcoreMesh(core_axis_name="c", subcore_axis_name="s"),
        out_type=jax.ShapeDtypeStruct((HIST, DIM, BATCH), jnp.float32),
        scratch_types=[
            pltpu.VMEM((BB, HC), jnp.int32),
            pltpu.VMEM((CHUNK,), jnp.int32),
            pltpu.VMEM((CHUNK, DIM), jnp.float32),
            pltpu.VMEM((HC, DIM, BB + 1), jnp.float32),
            pltpu.VMEM((2, DIM), jnp.float32),
            pltpu.SemaphoreType.DMA,
            pltpu.SemaphoreType.DMA,
        ],
        compiler_params=pltpu.CompilerParams(
            needs_layout_passes=False, use_tc_tiling_on_sc=False),
    )(_body)
    return f(idxes, table, beg_end)


def kernel(idxes, table, beg_end):
    return _run(idxes, table, beg_end).transpose(2, 0, 1)
